# baseline (device time: 2935419 ns/iter reference)
import jax
import jax.numpy as jnp
from jax import lax
from jax.experimental import pallas as pl
from jax.experimental.pallas import tpu as pltpu

N_DEV = 32


def _gelu(y):
    c = 0.7978845608028654
    return 0.5 * y * (1.0 + jnp.tanh(c * (y + 0.044715 * y * y * y)))


def kernel(x, w_mat):
    m_per, k = x.shape
    k2, n_per = w_mat.shape

    def body(x_ref, w_ref, out_ref, comm_ref, send_sems, recv_sems):
        my_pos = lax.axis_index("i")
        left = (my_pos - 1) % N_DEV
        right = (my_pos + 1) % N_DEV

        barrier_sem = pltpu.get_barrier_semaphore()
        for nbr in [left, right]:
            pl.semaphore_signal(
                barrier_sem, inc=1,
                device_id=(nbr,), device_id_type=pl.DeviceIdType.MESH,
            )
        pl.semaphore_wait(barrier_sem, 2)

        comm_ref[0, :, :] = x_ref[:, :]
        own = jnp.dot(x_ref[:, :], w_ref[:, :],
                      preferred_element_type=jnp.float32)
        out_ref[pl.ds(my_pos * m_per, m_per), :] = _gelu(own)

        for h in range(N_DEV - 1):
            send_slot = h % 2
            recv_slot = (h + 1) % 2
            rdma = pltpu.make_async_remote_copy(
                src_ref=comm_ref.at[send_slot],
                dst_ref=comm_ref.at[recv_slot],
                send_sem=send_sems.at[send_slot],
                recv_sem=recv_sems.at[recv_slot],
                device_id=(right,),
                device_id_type=pl.DeviceIdType.MESH,
            )
            rdma.start()
            rdma.wait()

            origin = (my_pos - h - 1) % N_DEV
            y = jnp.dot(comm_ref[recv_slot, :, :], w_ref[:, :],
                        preferred_element_type=jnp.float32)
            out_ref[pl.ds(origin * m_per, m_per), :] = _gelu(y)

    return pl.pallas_call(
        body,
        out_shape=jax.ShapeDtypeStruct((N_DEV * m_per, n_per), jnp.float32),
        in_specs=[
            pl.BlockSpec(memory_space=pltpu.VMEM),
            pl.BlockSpec(memory_space=pltpu.VMEM),
        ],
        out_specs=pl.BlockSpec(memory_space=pltpu.VMEM),
        scratch_shapes=[
            pltpu.VMEM((2, m_per, k), jnp.float32),
            pltpu.SemaphoreType.DMA((2,)),
            pltpu.SemaphoreType.DMA((2,)),
        ],
        compiler_params=pltpu.CompilerParams(collective_id=0),
    )(x, w_mat)


# device time: 1502854 ns/iter; 1.9532x vs baseline; 1.9532x over previous
import jax
import jax.numpy as jnp
import numpy as np
from jax import lax
from jax.experimental import pallas as pl
from jax.experimental.pallas import tpu as pltpu

N_DEV = 32
R_HOPS = N_DEV // 2
L_HOPS = N_DEV - 1 - R_HOPS

PERM = np.array(
    [0, 3, 4, 7, 15, 12, 11, 8, 16, 19, 20, 23, 31, 28, 27, 24,
     25, 26, 29, 30, 22, 21, 18, 17, 9, 10, 13, 14, 6, 5, 2, 1],
    dtype=np.int32,
)
INV = np.zeros(N_DEV, dtype=np.int32)
INV[PERM] = np.arange(N_DEV, dtype=np.int32)


def _gelu(y):
    c = 0.7978845608028654
    return 0.5 * y * (1.0 + jnp.tanh(c * (y + 0.044715 * y * y * y)))


def kernel(x, w_mat):
    m_per, k = x.shape
    k2, n_per = w_mat.shape

    my_log = lax.axis_index("i")
    perm = jnp.asarray(PERM)
    inv = jnp.asarray(INV)
    my_ring = inv[my_log]
    right = perm[(my_ring + 1) % N_DEV]
    left = perm[(my_ring - 1) % N_DEV]
    h_r = jnp.arange(1, R_HOPS + 1, dtype=jnp.int32)
    h_l = jnp.arange(1, L_HOPS + 1, dtype=jnp.int32)
    origins_r = perm[(my_ring - h_r) % N_DEV]
    origins_l = perm[(my_ring + h_l) % N_DEV]
    meta = jnp.concatenate(
        [jnp.stack([right, left, my_log]), origins_r, origins_l]
    ).astype(jnp.int32)

    def body(meta_ref, x_ref, w_ref, out_ref,
             comm_r, comm_l, send_r, recv_r, send_l, recv_l,
             credit_r, credit_l):
        right = meta_ref[0]
        left = meta_ref[1]
        my_log_ = meta_ref[2]

        def gemm_store(chunk, origin):
            y = jnp.dot(chunk, w_ref[:, :], preferred_element_type=jnp.float32)
            out_ref[pl.ds(origin * m_per, m_per), :] = _gelu(y)

        barrier_sem = pltpu.get_barrier_semaphore()
        for nbr in [left, right]:
            pl.semaphore_signal(
                barrier_sem, inc=1,
                device_id=(nbr,), device_id_type=pl.DeviceIdType.MESH,
            )
        pl.semaphore_wait(barrier_sem, 2)

        for h in range(R_HOPS):
            sr = h % 2
            rr = (h + 1) % 2
            if h > 0:
                pl.semaphore_wait(credit_r, 1)
            rdma_r = pltpu.make_async_remote_copy(
                src_ref=x_ref if h == 0 else comm_r.at[sr],
                dst_ref=comm_r.at[rr],
                send_sem=send_r.at[sr],
                recv_sem=recv_r.at[rr],
                device_id=(right,),
                device_id_type=pl.DeviceIdType.MESH,
            )
            rdma_r.start()
            if h < L_HOPS:
                if h > 0:
                    pl.semaphore_wait(credit_l, 1)
                rdma_l = pltpu.make_async_remote_copy(
                    src_ref=x_ref if h == 0 else comm_l.at[sr],
                    dst_ref=comm_l.at[rr],
                    send_sem=send_l.at[sr],
                    recv_sem=recv_l.at[rr],
                    device_id=(left,),
                    device_id_type=pl.DeviceIdType.MESH,
                )
                rdma_l.start()

            if h == 0:
                gemm_store(x_ref[:, :], my_log_)
            else:
                gemm_store(comm_r[sr, :, :], meta_ref[3 + (h - 1)])
                if h <= L_HOPS:
                    gemm_store(comm_l[sr, :, :], meta_ref[3 + R_HOPS + (h - 1)])

            rdma_r.wait()
            if h < L_HOPS:
                rdma_l.wait()

            if h < R_HOPS - 1:
                pl.semaphore_signal(
                    credit_r, inc=1,
                    device_id=(left,), device_id_type=pl.DeviceIdType.MESH,
                )
            if h < L_HOPS - 1:
                pl.semaphore_signal(
                    credit_l, inc=1,
                    device_id=(right,), device_id_type=pl.DeviceIdType.MESH,
                )

        gemm_store(comm_r[R_HOPS % 2, :, :], meta_ref[3 + R_HOPS - 1])

    return pl.pallas_call(
        body,
        out_shape=jax.ShapeDtypeStruct((N_DEV * m_per, n_per), jnp.float32),
        in_specs=[
            pl.BlockSpec(memory_space=pltpu.SMEM),
            pl.BlockSpec(memory_space=pltpu.VMEM),
            pl.BlockSpec(memory_space=pltpu.VMEM),
        ],
        out_specs=pl.BlockSpec(memory_space=pltpu.VMEM),
        scratch_shapes=[
            pltpu.VMEM((2, m_per, k), jnp.float32),
            pltpu.VMEM((2, m_per, k), jnp.float32),
            pltpu.SemaphoreType.DMA((2,)),
            pltpu.SemaphoreType.DMA((2,)),
            pltpu.SemaphoreType.DMA((2,)),
            pltpu.SemaphoreType.DMA((2,)),
            pltpu.SemaphoreType.REGULAR,
            pltpu.SemaphoreType.REGULAR,
        ],
        compiler_params=pltpu.CompilerParams(
            collective_id=0,
            vmem_limit_bytes=100 * 1024 * 1024,
        ),
    )(meta, x, w_mat)


# device time: 1491055 ns/iter; 1.9687x vs baseline; 1.0079x over previous
import jax
import jax.numpy as jnp
import numpy as np
from jax import lax
from jax.experimental import pallas as pl
from jax.experimental.pallas import tpu as pltpu

N_DEV = 32
R_HOPS = N_DEV // 2
L_HOPS = N_DEV - 1 - R_HOPS
NSLOT = 3

PERM = np.array(
    [0, 3, 4, 7, 15, 12, 11, 8, 16, 19, 20, 23, 31, 28, 27, 24,
     25, 26, 29, 30, 22, 21, 18, 17, 9, 10, 13, 14, 6, 5, 2, 1],
    dtype=np.int32,
)
INV = np.zeros(N_DEV, dtype=np.int32)
INV[PERM] = np.arange(N_DEV, dtype=np.int32)


def _gelu(y):
    c = 0.7978845608028654
    return 0.5 * y * (1.0 + jnp.tanh(c * (y + 0.044715 * y * y * y)))


def kernel(x, w_mat):
    m_per, k = x.shape
    k2, n_per = w_mat.shape

    my_log = lax.axis_index("i")
    perm = jnp.asarray(PERM)
    inv = jnp.asarray(INV)
    my_ring = inv[my_log]
    right = perm[(my_ring + 1) % N_DEV]
    left = perm[(my_ring - 1) % N_DEV]
    h_r = jnp.arange(1, R_HOPS + 1, dtype=jnp.int32)
    h_l = jnp.arange(1, L_HOPS + 1, dtype=jnp.int32)
    origins_r = perm[(my_ring - h_r) % N_DEV]
    origins_l = perm[(my_ring + h_l) % N_DEV]
    meta = jnp.concatenate(
        [jnp.stack([right, left, my_log]), origins_r, origins_l]
    ).astype(jnp.int32)

    def body(meta_ref, x_ref, w_ref, out_ref,
             comm_r, comm_l, send_r, recv_r, send_l, recv_l,
             credit_r, credit_l):
        right = meta_ref[0]
        left = meta_ref[1]
        my_log_ = meta_ref[2]

        def gemm_store(chunk, origin):
            y = jnp.dot(chunk, w_ref[:, :], preferred_element_type=jnp.float32)
            out_ref[pl.ds(origin * m_per, m_per), :] = _gelu(y)

        barrier_sem = pltpu.get_barrier_semaphore()
        for nbr in [left, right]:
            pl.semaphore_signal(
                barrier_sem, inc=1,
                device_id=(nbr,), device_id_type=pl.DeviceIdType.MESH,
            )
        pl.semaphore_wait(barrier_sem, 2)

        def send_desc(comm, send_sems, recv_sems, h, nbr, nslot=NSLOT):
            return pltpu.make_async_remote_copy(
                src_ref=x_ref if h == 0 else comm.at[h % nslot],
                dst_ref=comm.at[(h + 1) % nslot],
                send_sem=send_sems.at[h % nslot],
                recv_sem=recv_sems.at[(h + 1) % nslot],
                device_id=(nbr,),
                device_id_type=pl.DeviceIdType.MESH,
            )

        def recv_wait(comm, send_sems, recv_sems, h, nbr, nslot=NSLOT):
            pltpu.make_async_remote_copy(
                src_ref=comm.at[(h + 1) % nslot],
                dst_ref=comm.at[(h + 1) % nslot],
                send_sem=send_sems.at[0],
                recv_sem=recv_sems.at[(h + 1) % nslot],
                device_id=(nbr,),
                device_id_type=pl.DeviceIdType.MESH,
            ).wait_recv()

        sends_r = {}
        sends_l = {}
        for h in range(R_HOPS):
            if h >= 1:
                recv_wait(comm_r, send_r, recv_r, h - 1, right)
            if h >= 3:
                pl.semaphore_wait(credit_r, 1)
            sends_r[h] = send_desc(comm_r, send_r, recv_r, h, right)
            sends_r[h].start()

            if h >= 1:
                recv_wait(comm_l, send_l, recv_l, h - 1, left, nslot=2)
                sends_l[h - 1].wait_send()
            if 2 <= h <= L_HOPS - 1:
                pl.semaphore_signal(
                    credit_l, inc=1,
                    device_id=(right,), device_id_type=pl.DeviceIdType.MESH,
                )
            if h <= L_HOPS - 1:
                if h >= 2:
                    pl.semaphore_wait(credit_l, 1)
                sends_l[h] = send_desc(comm_l, send_l, recv_l, h, left,
                                       nslot=2)
                sends_l[h].start()

            if h >= 1:
                sends_r[h - 1].wait_send()
            if 2 <= h <= R_HOPS - 2:
                pl.semaphore_signal(
                    credit_r, inc=1,
                    device_id=(left,), device_id_type=pl.DeviceIdType.MESH,
                )

            if h == 0:
                gemm_store(x_ref[:, :], my_log_)
            else:
                gemm_store(comm_r[h % NSLOT, :, :], meta_ref[3 + (h - 1)])
                gemm_store(comm_l[h % 2, :, :],
                           meta_ref[3 + R_HOPS + (h - 1)])

        recv_wait(comm_r, send_r, recv_r, R_HOPS - 1, right)
        gemm_store(comm_r[R_HOPS % NSLOT, :, :], meta_ref[3 + R_HOPS - 1])
        sends_r[R_HOPS - 1].wait_send()

    return pl.pallas_call(
        body,
        out_shape=jax.ShapeDtypeStruct((N_DEV * m_per, n_per), jnp.float32),
        in_specs=[
            pl.BlockSpec(memory_space=pltpu.SMEM),
            pl.BlockSpec(memory_space=pltpu.VMEM),
            pl.BlockSpec(memory_space=pltpu.VMEM),
        ],
        out_specs=pl.BlockSpec(memory_space=pltpu.VMEM),
        scratch_shapes=[
            pltpu.VMEM((NSLOT, m_per, k), jnp.float32),
            pltpu.VMEM((2, m_per, k), jnp.float32),
            pltpu.SemaphoreType.DMA((NSLOT,)),
            pltpu.SemaphoreType.DMA((NSLOT,)),
            pltpu.SemaphoreType.DMA((2,)),
            pltpu.SemaphoreType.DMA((2,)),
            pltpu.SemaphoreType.REGULAR,
            pltpu.SemaphoreType.REGULAR,
        ],
        compiler_params=pltpu.CompilerParams(
            collective_id=0,
            vmem_limit_bytes=100 * 1024 * 1024,
        ),
    )(meta, x, w_mat)


# device time: 1455189 ns/iter; 2.0172x vs baseline; 1.0246x over previous
import jax
import jax.numpy as jnp
import numpy as np
from jax import lax
from jax.experimental import pallas as pl
from jax.experimental.pallas import tpu as pltpu

N_DEV = 32
R_HOPS = N_DEV // 2
L_HOPS = N_DEV - 1 - R_HOPS
NSLOT = 3

PERM = np.array(
    [0, 3, 4, 7, 15, 12, 11, 8, 16, 19, 20, 23, 31, 28, 27, 24,
     25, 26, 29, 30, 22, 21, 18, 17, 9, 10, 13, 14, 6, 5, 2, 1],
    dtype=np.int32,
)
INV = np.zeros(N_DEV, dtype=np.int32)
INV[PERM] = np.arange(N_DEV, dtype=np.int32)


def _gelu(y):
    c = 0.7978845608028654
    return 0.5 * y * (1.0 + jnp.tanh(c * (y + 0.044715 * y * y * y)))


def kernel(x, w_mat):
    m_per, k = x.shape
    k2, n_per = w_mat.shape

    my_log = lax.axis_index("i")
    perm = jnp.asarray(PERM)
    inv = jnp.asarray(INV)
    my_ring = inv[my_log]
    right = perm[(my_ring + 1) % N_DEV]
    left = perm[(my_ring - 1) % N_DEV]
    h_r = jnp.arange(1, R_HOPS + 1, dtype=jnp.int32)
    h_l = jnp.arange(1, L_HOPS + 1, dtype=jnp.int32)
    origins_r = perm[(my_ring - h_r) % N_DEV]
    origins_l = perm[(my_ring + h_l) % N_DEV]
    meta = jnp.concatenate(
        [jnp.stack([right, left, my_log]), origins_r, origins_l]
    ).astype(jnp.int32)

    def body(meta_ref, x_ref, w_ref, out_ref,
             comm_r, comm_l, send_r, recv_r, send_l, recv_l,
             credit_r, credit_l):
        right = meta_ref[0]
        left = meta_ref[1]
        my_log_ = meta_ref[2]

        def gemm_store(chunk, origin):
            y = jnp.dot(chunk, w_ref[:, :], preferred_element_type=jnp.float32)
            out_ref[pl.ds(origin * m_per, m_per), :] = _gelu(y)

        barrier_sem = pltpu.get_barrier_semaphore()
        for nbr in [left, right]:
            pl.semaphore_signal(
                barrier_sem, inc=1,
                device_id=(nbr,), device_id_type=pl.DeviceIdType.MESH,
            )
        pl.semaphore_wait(barrier_sem, 2)

        def send_desc(comm, send_sems, recv_sems, h, nbr, nslot=NSLOT):
            return pltpu.make_async_remote_copy(
                src_ref=x_ref if h == 0 else comm.at[h % nslot],
                dst_ref=comm.at[(h + 1) % nslot],
                send_sem=send_sems.at[h % nslot],
                recv_sem=recv_sems.at[(h + 1) % nslot],
                device_id=(nbr,),
                device_id_type=pl.DeviceIdType.MESH,
            )

        def recv_wait(comm, send_sems, recv_sems, h, nbr, nslot=NSLOT):
            pltpu.make_async_remote_copy(
                src_ref=comm.at[(h + 1) % nslot],
                dst_ref=comm.at[(h + 1) % nslot],
                send_sem=send_sems.at[0],
                recv_sem=recv_sems.at[(h + 1) % nslot],
                device_id=(nbr,),
                device_id_type=pl.DeviceIdType.MESH,
            ).wait_recv()

        sends_r = {}
        sends_l = {}
        for h in range(R_HOPS - 1):
            if h >= 1:
                recv_wait(comm_r, send_r, recv_r, h - 1, right)
            if h >= 3:
                pl.semaphore_wait(credit_r, 1)
            sends_r[h] = send_desc(comm_r, send_r, recv_r, h, right)
            sends_r[h].start()

            if h >= 1:
                recv_wait(comm_l, send_l, recv_l, h - 1, left, nslot=2)
                sends_l[h - 1].wait_send()
            if 2 <= h <= L_HOPS - 1:
                pl.semaphore_signal(
                    credit_l, inc=1,
                    device_id=(right,), device_id_type=pl.DeviceIdType.MESH,
                )
            if h >= 2:
                pl.semaphore_wait(credit_l, 1)
            sends_l[h] = send_desc(comm_l, send_l, recv_l, h, left, nslot=2)
            sends_l[h].start()

            if h >= 1:
                sends_r[h - 1].wait_send()
            if 2 <= h <= R_HOPS - 2:
                pl.semaphore_signal(
                    credit_r, inc=1,
                    device_id=(left,), device_id_type=pl.DeviceIdType.MESH,
                )

            if h == 0:
                gemm_store(x_ref[:, :], my_log_)
            else:
                gemm_store(comm_r[h % NSLOT, :, :], meta_ref[3 + (h - 1)])
                gemm_store(comm_l[h % 2, :, :],
                           meta_ref[3 + R_HOPS + (h - 1)])

        hh = R_HOPS - 1
        half = m_per // 2
        recv_wait(comm_r, send_r, recv_r, hh - 1, right)
        pl.semaphore_wait(credit_r, 1)
        half_r = pltpu.make_async_remote_copy(
            src_ref=comm_r.at[hh % NSLOT, pl.ds(0, half), :],
            dst_ref=comm_r.at[(hh + 1) % NSLOT, pl.ds(0, half), :],
            send_sem=send_r.at[hh % NSLOT],
            recv_sem=recv_r.at[(hh + 1) % NSLOT],
            device_id=(right,),
            device_id_type=pl.DeviceIdType.MESH,
        )
        half_r.start()

        recv_wait(comm_l, send_l, recv_l, hh - 1, left, nslot=2)
        sends_l[hh - 1].wait_send()
        pl.semaphore_signal(
            credit_l, inc=1,
            device_id=(right,), device_id_type=pl.DeviceIdType.MESH,
        )
        pl.semaphore_wait(credit_l, 1)
        half_l = pltpu.make_async_remote_copy(
            src_ref=comm_l.at[hh % 2, pl.ds(half, half), :],
            dst_ref=comm_l.at[(hh + 1) % 2, pl.ds(half, half), :],
            send_sem=send_l.at[hh % 2],
            recv_sem=recv_l.at[(hh + 1) % 2],
            device_id=(left,),
            device_id_type=pl.DeviceIdType.MESH,
        )
        half_l.start()

        sends_r[hh - 1].wait_send()
        gemm_store(comm_r[hh % NSLOT, :, :], meta_ref[3 + (hh - 1)])
        gemm_store(comm_l[hh % 2, :, :], meta_ref[3 + R_HOPS + (hh - 1)])

        origin_a = meta_ref[3 + R_HOPS - 1]
        pltpu.make_async_remote_copy(
            src_ref=comm_r.at[(hh + 1) % NSLOT, pl.ds(0, half), :],
            dst_ref=comm_r.at[(hh + 1) % NSLOT, pl.ds(0, half), :],
            send_sem=send_r.at[0],
            recv_sem=recv_r.at[(hh + 1) % NSLOT],
            device_id=(right,),
            device_id_type=pl.DeviceIdType.MESH,
        ).wait_recv()
        y_top = jnp.dot(comm_r[(hh + 1) % NSLOT, :half, :], w_ref[:, :],
                        preferred_element_type=jnp.float32)
        out_ref[pl.ds(origin_a * m_per, half), :] = _gelu(y_top)

        pltpu.make_async_remote_copy(
            src_ref=comm_l.at[(hh + 1) % 2, pl.ds(half, half), :],
            dst_ref=comm_l.at[(hh + 1) % 2, pl.ds(half, half), :],
            send_sem=send_l.at[0],
            recv_sem=recv_l.at[(hh + 1) % 2],
            device_id=(left,),
            device_id_type=pl.DeviceIdType.MESH,
        ).wait_recv()
        y_bot = jnp.dot(comm_l[(hh + 1) % 2, half:, :], w_ref[:, :],
                        preferred_element_type=jnp.float32)
        out_ref[pl.ds(origin_a * m_per + half, half), :] = _gelu(y_bot)

        half_r.wait_send()
        half_l.wait_send()

    return pl.pallas_call(
        body,
        out_shape=jax.ShapeDtypeStruct((N_DEV * m_per, n_per), jnp.float32),
        in_specs=[
            pl.BlockSpec(memory_space=pltpu.SMEM),
            pl.BlockSpec(memory_space=pltpu.VMEM),
            pl.BlockSpec(memory_space=pltpu.VMEM),
        ],
        out_specs=pl.BlockSpec(memory_space=pltpu.VMEM),
        scratch_shapes=[
            pltpu.VMEM((NSLOT, m_per, k), jnp.float32),
            pltpu.VMEM((2, m_per, k), jnp.float32),
            pltpu.SemaphoreType.DMA((NSLOT,)),
            pltpu.SemaphoreType.DMA((NSLOT,)),
            pltpu.SemaphoreType.DMA((2,)),
            pltpu.SemaphoreType.DMA((2,)),
            pltpu.SemaphoreType.REGULAR,
            pltpu.SemaphoreType.REGULAR,
        ],
        compiler_params=pltpu.CompilerParams(
            collective_id=0,
            vmem_limit_bytes=100 * 1024 * 1024,
        ),
    )(meta, x, w_mat)
